# Initial kernel scaffold; baseline (speedup 1.0000x reference)
import jax
import jax.numpy as jnp
from jax.experimental import pallas as pl


def kernel(x, table, bias):
    raise NotImplementedError("write your pallas kernel here")


# speedup vs baseline: 1.2876x; 1.2876x over previous
import functools

import jax
import jax.numpy as jnp
from jax import lax
from jax.experimental import pallas as pl
from jax.experimental.pallas import tpu as pltpu
from jax.experimental.pallas import tpu_sc as plsc

_NC = 2
_NS = 16
_LANES = 16
_NW = _NC * _NS


@functools.lru_cache(maxsize=None)
def _build(batch: int, n_fields: int):
    n_idx = batch * n_fields
    assert n_idx % _NW == 0 and batch % (_NW * _LANES) == 0
    n_per_w = n_idx // _NW
    s_per_w = batch // _NW
    mesh = plsc.VectorSubcoreMesh(core_axis_name="c", subcore_axis_name="s")

    @functools.partial(
        pl.kernel,
        mesh=mesh,
        out_type=jax.ShapeDtypeStruct((batch,), jnp.float32),
        compiler_params=pltpu.CompilerParams(needs_layout_passes=False),
        scratch_types=[
            pltpu.VMEM((n_per_w,), jnp.int32),
            pltpu.VMEM((n_per_w,), jnp.float32),
            pltpu.VMEM((_LANES,), jnp.float32),
            pltpu.VMEM((s_per_w,), jnp.float32),
            pltpu.SemaphoreType.DMA,
        ],
    )
    def lr_kernel(idx_hbm, tab_hbm, bias_hbm, out_hbm,
                  idx_v, vals_v, bias_v, out_v, sem):
        wid = lax.axis_index("s") * _NC + lax.axis_index("c")
        pltpu.sync_copy(idx_hbm.at[pl.ds(wid * n_per_w, n_per_w)], idx_v)
        pltpu.sync_copy(bias_hbm, bias_v)
        pltpu.async_copy(tab_hbm.at[idx_v], vals_v, sem).wait()
        bias_vec = bias_v[...]

        def group(g, carry):
            sl = g * _LANES + lax.iota(jnp.int32, _LANES)
            base = sl * n_fields
            acc = bias_vec
            for j in range(n_fields):
                acc = acc + plsc.load_gather(vals_v, [base + j])
            sig = 1.0 / (1.0 + jnp.exp(-acc))
            plsc.store_scatter(out_v, [sl], sig)
            return carry

        lax.fori_loop(0, s_per_w // _LANES, group, 0)
        pltpu.sync_copy(out_v, out_hbm.at[pl.ds(wid * s_per_w, s_per_w)])

    return lr_kernel


def kernel(x, table, bias):
    batch, n_fields = x.shape
    idx_flat = x.reshape(-1).astype(jnp.int32)
    tab_flat = table.reshape(-1).astype(jnp.float32)
    bias16 = jnp.broadcast_to(
        bias.reshape(-1)[:1], (_LANES,)).astype(jnp.float32)
    out = _build(batch, n_fields)(idx_flat, tab_flat, bias16)
    return out.reshape(batch, 1)


# speedup vs baseline: 1.3443x; 1.0440x over previous
import functools

import jax
import jax.numpy as jnp
from jax import lax
from jax.experimental import pallas as pl
from jax.experimental.pallas import tpu as pltpu
from jax.experimental.pallas import tpu_sc as plsc

_NC = 2
_NS = 16
_LANES = 16
_NW = _NC * _NS


@functools.lru_cache(maxsize=None)
def _build(batch: int, n_fields: int):
    n_idx = batch * n_fields
    assert n_idx % _NW == 0 and batch % (_NW * _LANES) == 0
    n_per_w = n_idx // _NW
    s_per_w = batch // _NW
    mesh = plsc.VectorSubcoreMesh(core_axis_name="c", subcore_axis_name="s")

    @functools.partial(
        pl.kernel,
        mesh=mesh,
        out_type=jax.ShapeDtypeStruct((batch,), jnp.float32),
        compiler_params=pltpu.CompilerParams(needs_layout_passes=False),
        scratch_types=[
            pltpu.VMEM((s_per_w, n_fields), jnp.int32),
            pltpu.VMEM((n_per_w,), jnp.int32),
            pltpu.VMEM((n_per_w,), jnp.float32),
            pltpu.VMEM((_LANES,), jnp.float32),
            pltpu.VMEM((s_per_w,), jnp.float32),
            pltpu.SemaphoreType.DMA,
        ],
    )
    def lr_kernel(x_hbm, tab_hbm, bias_hbm, out_hbm,
                  x_v, idx_v, vals_v, bias_v, out_v, sem):
        wid = lax.axis_index("s") * _NC + lax.axis_index("c")
        pltpu.sync_copy(x_hbm.at[pl.ds(wid * s_per_w, s_per_w), :], x_v)
        pltpu.sync_copy(bias_hbm, bias_v)

        def flat(g, carry):
            p = g * _LANES + lax.iota(jnp.int32, _LANES)
            sl = p // n_fields
            j = p - sl * n_fields
            plsc.store_scatter(idx_v, [p], plsc.load_gather(x_v, [sl, j]))
            return carry

        lax.fori_loop(0, n_per_w // _LANES, flat, 0)
        pltpu.async_copy(tab_hbm.at[idx_v], vals_v, sem).wait()
        bias_vec = bias_v[...]

        def group(g, carry):
            sl = g * _LANES + lax.iota(jnp.int32, _LANES)
            base = sl * n_fields
            acc = bias_vec
            for j in range(n_fields):
                acc = acc + plsc.load_gather(vals_v, [base + j])
            sig = 1.0 / (1.0 + jnp.exp(-acc))
            plsc.store_scatter(out_v, [sl], sig)
            return carry

        lax.fori_loop(0, s_per_w // _LANES, group, 0)
        pltpu.sync_copy(out_v, out_hbm.at[pl.ds(wid * s_per_w, s_per_w)])

    return lr_kernel


def kernel(x, table, bias):
    batch, n_fields = x.shape
    tab_flat = table.astype(jnp.float32).reshape(-1)
    bias16 = jnp.broadcast_to(
        bias.reshape(-1)[:1], (_LANES,)).astype(jnp.float32)
    out = _build(batch, n_fields)(x.astype(jnp.int32), tab_flat, bias16)
    return out.reshape(batch, 1)
